# baseline (speedup 1.0000x reference)
import functools

import jax
import jax.numpy as jnp
from jax import lax
from jax.experimental import pallas as pl
from jax.experimental.pallas import tpu as pltpu
from jax.experimental.pallas import tpu_sc as plsc

N = 10000
F = 128
E = 320000
NCLS = 919

NC = 2
NS = 16
NW = NC * NS
CH = 64
CPW = 160
EP = NW * CPW * CH
NBUF = 5
NIDX = 2 * NBUF
NGRP = CPW // NIDX
BR = 80
NB = N // BR
BPT = -(-NB // NS)

BN = 1000


def _sc_agg_body(support_hbm, src_hbm, dst_hbm, zeros_hbm, out_hbm,
                 idx_s, idx_d, rows_v, acc, *sems):
    isems = sems[:NIDX]
    gsems = sems[NIDX:NIDX + NBUF]
    ssems = sems[NIDX + NBUF:]
    c = lax.axis_index("c")
    s = lax.axis_index("s")
    wid = c * NS + s

    def idx_start(chunk, q):
        pltpu.async_copy(src_hbm.at[wid, chunk], idx_s.at[q], isems[q])
        pltpu.async_copy(dst_hbm.at[wid, chunk], idx_d.at[q], isems[q])

    def idx_wait(q):
        pltpu.make_async_copy(src_hbm.at[wid, 0], idx_s.at[q],
                              isems[q]).wait()
        pltpu.make_async_copy(dst_hbm.at[wid, 0], idx_d.at[q],
                              isems[q]).wait()

    def gather_start(q, b):
        pltpu.async_copy(support_hbm.at[idx_s.at[q]], rows_v.at[b], gsems[b])

    def gather_wait(q, b):
        pltpu.make_async_copy(support_hbm.at[idx_s.at[q]], rows_v.at[b],
                              gsems[b]).wait()

    def scatter_start(q, b):
        pltpu.async_copy(rows_v.at[b], acc.at[idx_d.at[q]], ssems[b],
                         add=True)

    def scatter_wait(q, b):
        pltpu.make_async_copy(rows_v.at[b], acc.at[idx_d.at[q]],
                              ssems[b]).wait()

    for k in range(NBUF):
        idx_start(k, k)

    def zero_blk(i, _):
        j = i * NS + s

        @pl.when(j < NB)
        def _():
            pltpu.sync_copy(zeros_hbm, acc.at[pl.ds(j * BR, BR), :])
        return 0

    lax.fori_loop(0, BPT, zero_blk, 0)
    plsc.subcore_barrier()

    idx_wait(0)
    gather_start(0, 0)

    def outer(i0, _):
        for k in range(NIDX):
            i = i0 * NIDX + k
            b = k % NBUF
            q = k
            q1 = (k + 1) % NIDX
            b1 = (k + 1) % NBUF

            gather_wait(q, b)
            scatter_start(q, b)

            @pl.when(i + 1 < CPW)
            def _():
                idx_wait(q1)

                @pl.when(i + 1 >= NBUF)
                def _():
                    scatter_wait(q1, b1)
                gather_start(q1, b1)

            @pl.when(i + NBUF < CPW)
            def _():
                idx_start(i + NBUF, (k + NBUF) % NIDX)
        return 0

    lax.fori_loop(0, NGRP, outer, 0)

    for j in range(CPW - NBUF, CPW):
        scatter_wait(j % NIDX, j % NBUF)
    plsc.subcore_barrier()

    def drain_blk(i, _):
        j = i * NS + s

        @pl.when(j < NB)
        def _():
            pltpu.sync_copy(acc.at[pl.ds(j * BR, BR), :],
                            out_hbm.at[c, pl.ds(j * BR, BR), :])
        return 0

    lax.fori_loop(0, BPT, drain_blk, 0)


_sc_agg = functools.partial(
    pl.kernel,
    out_type=jax.ShapeDtypeStruct((NC, N, F), jnp.float32),
    mesh=plsc.VectorSubcoreMesh(
        core_axis_name="c", subcore_axis_name="s", num_cores=NC,
        num_subcores=NS),
    scratch_types=(
        [
            pltpu.VMEM((NIDX, CH), jnp.int32),
            pltpu.VMEM((NIDX, CH), jnp.int32),
            pltpu.VMEM((NBUF, CH, F), jnp.float32),
            pltpu.VMEM_SHARED((N + 8, F), jnp.float32),
        ]
        + [pltpu.SemaphoreType.DMA] * (NIDX + 2 * NBUF)
    ),
)(_sc_agg_body)


def _mm_body(x_ref, w_ref, o_ref):
    o_ref[...] = jnp.dot(x_ref[...], w_ref[...],
                         preferred_element_type=jnp.float32)


def _support1(x, W):
    return pl.pallas_call(
        _mm_body,
        grid=(N // BN,),
        in_specs=[
            pl.BlockSpec((BN, F), lambda i: (i, 0)),
            pl.BlockSpec((F, F), lambda i: (0, 0)),
        ],
        out_specs=pl.BlockSpec((BN, F), lambda i: (i, 0)),
        out_shape=jax.ShapeDtypeStruct((N, F), jnp.float32),
    )(x, W)


def _gate_body(p_ref, q_ref, deg_ref, b1_ref, wg_ref, bg_ref, x_ref, w2_ref,
               g_ref, x1_ref, s2_ref):
    agg = (p_ref[0] + q_ref[0]) / deg_ref[...] + b1_ref[...]
    z = jnp.tanh(agg)
    g = jax.nn.sigmoid(jnp.dot(z, wg_ref[...],
                               preferred_element_type=jnp.float32)
                       + bg_ref[...])
    x1 = (1.0 - g) * x_ref[...] + g * z
    g_ref[...] = g
    x1_ref[...] = x1
    s2_ref[...] = jnp.dot(x1, w2_ref[...], preferred_element_type=jnp.float32)


def _gate_stage(parts, deg, b1, wg, bg, x, W2):
    return pl.pallas_call(
        _gate_body,
        grid=(N // BN,),
        in_specs=[
            pl.BlockSpec((1, BN, F), lambda i: (0, i, 0)),
            pl.BlockSpec((1, BN, F), lambda i: (1, i, 0)),
            pl.BlockSpec((BN, 1), lambda i: (i, 0)),
            pl.BlockSpec((1, F), lambda i: (0, 0)),
            pl.BlockSpec((F, 1), lambda i: (0, 0)),
            pl.BlockSpec((1, 1), lambda i: (0, 0)),
            pl.BlockSpec((BN, F), lambda i: (i, 0)),
            pl.BlockSpec((F, F), lambda i: (0, 0)),
        ],
        out_specs=[
            pl.BlockSpec((BN, 1), lambda i: (i, 0)),
            pl.BlockSpec((BN, F), lambda i: (i, 0)),
            pl.BlockSpec((BN, F), lambda i: (i, 0)),
        ],
        out_shape=[
            jax.ShapeDtypeStruct((N, 1), jnp.float32),
            jax.ShapeDtypeStruct((N, F), jnp.float32),
            jax.ShapeDtypeStruct((N, F), jnp.float32),
        ],
    )(parts, parts, deg, b1, wg, bg, x, W2)


def _final_body(p_ref, q_ref, deg_ref, b2_ref, wg_ref, bg_ref, x_ref,
                mu_ref, isg_ref, beta_ref, wo_ref, bo_ref,
                g_ref, out_ref):
    agg = (p_ref[0] + q_ref[0]) / deg_ref[...] + b2_ref[...]
    z = jnp.tanh(agg)
    g = jax.nn.sigmoid(jnp.dot(z, wg_ref[...],
                               preferred_element_type=jnp.float32)
                       + bg_ref[...])
    x2 = (1.0 - g) * x_ref[...] + g * z
    x2 = jnp.maximum(x2, 0.0)
    xb = (x2 - mu_ref[...]) * isg_ref[...] + beta_ref[...]
    g_ref[...] = g
    out_ref[...] = jnp.dot(xb, wo_ref[...],
                           preferred_element_type=jnp.float32) + bo_ref[...]


def _final_stage(parts, deg, b2, wg, bg, x1, mu, isg, beta, Wo, bo):
    return pl.pallas_call(
        _final_body,
        grid=(N // BN,),
        in_specs=[
            pl.BlockSpec((1, BN, F), lambda i: (0, i, 0)),
            pl.BlockSpec((1, BN, F), lambda i: (1, i, 0)),
            pl.BlockSpec((BN, 1), lambda i: (i, 0)),
            pl.BlockSpec((1, F), lambda i: (0, 0)),
            pl.BlockSpec((F, 1), lambda i: (0, 0)),
            pl.BlockSpec((1, 1), lambda i: (0, 0)),
            pl.BlockSpec((BN, F), lambda i: (i, 0)),
            pl.BlockSpec((1, F), lambda i: (0, 0)),
            pl.BlockSpec((1, F), lambda i: (0, 0)),
            pl.BlockSpec((1, F), lambda i: (0, 0)),
            pl.BlockSpec((F, NCLS), lambda i: (0, 0)),
            pl.BlockSpec((1, NCLS), lambda i: (0, 0)),
        ],
        out_specs=[
            pl.BlockSpec((BN, 1), lambda i: (i, 0)),
            pl.BlockSpec((BN, NCLS), lambda i: (i, 0)),
        ],
        out_shape=[
            jax.ShapeDtypeStruct((N, 1), jnp.float32),
            jax.ShapeDtypeStruct((N, NCLS), jnp.float32),
        ],
    )(parts, parts, deg, b2, wg, bg, x1, mu, isg, beta, Wo, bo)


def kernel(x_in, edge_index, deg, W_gc1, b_gc1, w_g1, b_g1, W_gc2, b_gc2,
           w_g2, b_g2, bn_gamma, bn_beta, bn_mean, bn_var, W_out, b_out):
    src = jnp.concatenate(
        [edge_index[0], jnp.zeros((EP - E,), jnp.int32)]).reshape(NW, CPW, CH)
    dst = jnp.concatenate(
        [edge_index[1], jnp.full((EP - E,), N, jnp.int32)]).reshape(
            NW, CPW, CH)
    zeros = jnp.zeros((BR, F), jnp.float32)

    support1 = _support1(x_in, W_gc1)
    parts1 = _sc_agg(support1, src, dst, zeros)
    g, x1, support2 = _gate_stage(
        parts1, deg, b_gc1.reshape(1, F), w_g1, b_g1.reshape(1, 1),
        x_in, W_gc2)

    parts2 = _sc_agg(support2, src, dst, zeros)
    inv_sigma = (bn_gamma / jnp.sqrt(bn_var + 1e-5)).reshape(1, F)
    g2, out = _final_stage(
        parts2, deg, b_gc2.reshape(1, F), w_g2, b_g2.reshape(1, 1),
        x1, bn_mean.reshape(1, F), inv_sigma, bn_beta.reshape(1, F),
        W_out, b_out.reshape(1, NCLS))

    return (x_in, out, g, g2)


# speedup vs baseline: 1.1119x; 1.1119x over previous
import functools

import jax
import jax.numpy as jnp
from jax import lax
from jax.experimental import pallas as pl
from jax.experimental.pallas import tpu as pltpu
from jax.experimental.pallas import tpu_sc as plsc

N = 10000
F = 128
E = 320000
NCLS = 919

NC = 2
NS = 16
NW = NC * NS
CH = 64
CPW = 160
EP = NW * CPW * CH
NBUF = 5
NIDX = 2 * NBUF
NGRP = CPW // NIDX
BR = 80
NB = N // BR
BPT = -(-NB // NS)

BN = 1000


def _sc_agg_body(support_hbm, src_hbm, dst_hbm, zeros_hbm, out_hbm,
                 idx_s, idx_d, rows_v, acc, *sems):
    isems = sems[:NIDX]
    gsems = sems[NIDX:NIDX + NBUF]
    ssems = sems[NIDX + NBUF:]
    c = lax.axis_index("c")
    s = lax.axis_index("s")
    wid = c * NS + s

    def idx_start(chunk, q):
        pltpu.async_copy(src_hbm.at[wid, chunk], idx_s.at[q], isems[q])
        pltpu.async_copy(dst_hbm.at[wid, chunk], idx_d.at[q], isems[q])

    def idx_wait(q):
        pltpu.make_async_copy(src_hbm.at[wid, 0], idx_s.at[q],
                              isems[q]).wait()
        pltpu.make_async_copy(dst_hbm.at[wid, 0], idx_d.at[q],
                              isems[q]).wait()

    def gather_start(q, b):
        pltpu.async_copy(support_hbm.at[idx_s.at[q]], rows_v.at[b], gsems[b])

    def gather_wait(q, b):
        pltpu.make_async_copy(support_hbm.at[idx_s.at[q]], rows_v.at[b],
                              gsems[b]).wait()

    def scatter_start(q, b):
        pltpu.async_copy(rows_v.at[b], acc.at[idx_d.at[q]], ssems[b],
                         add=True)

    def scatter_wait(q, b):
        pltpu.make_async_copy(rows_v.at[b], acc.at[idx_d.at[q]],
                              ssems[b]).wait()

    for k in range(NBUF):
        idx_start(k, k)

    def zero_blk(i, _):
        j = i * NS + s

        @pl.when(j < NB)
        def _():
            pltpu.sync_copy(zeros_hbm, acc.at[pl.ds(j * BR, BR), :])
        return 0

    lax.fori_loop(0, BPT, zero_blk, 0)
    plsc.subcore_barrier()

    idx_wait(0)
    gather_start(0, 0)

    def outer(i0, _):
        for k in range(NIDX):
            i = i0 * NIDX + k
            b = k % NBUF
            q = k
            q1 = (k + 1) % NIDX
            b1 = (k + 1) % NBUF

            gather_wait(q, b)
            scatter_start(q, b)

            @pl.when(i + 1 < CPW)
            def _():
                idx_wait(q1)

                @pl.when(i + 1 >= NBUF)
                def _():
                    scatter_wait(q1, b1)
                gather_start(q1, b1)

            @pl.when(i + NBUF < CPW)
            def _():
                idx_start(i + NBUF, (k + NBUF) % NIDX)
        return 0

    lax.fori_loop(0, NGRP, outer, 0)

    for j in range(CPW - NBUF, CPW):
        scatter_wait(j % NIDX, j % NBUF)
    plsc.subcore_barrier()

    def drain_blk(i, _):
        j = i * NS + s

        @pl.when(j < NB)
        def _():
            pltpu.sync_copy(acc.at[pl.ds(j * BR, BR), :],
                            out_hbm.at[c, pl.ds(j * BR, BR), :])
        return 0

    lax.fori_loop(0, BPT, drain_blk, 0)


_sc_agg = functools.partial(
    pl.kernel,
    out_type=jax.ShapeDtypeStruct((NC, N, F), jnp.float32),
    mesh=plsc.VectorSubcoreMesh(
        core_axis_name="c", subcore_axis_name="s", num_cores=NC,
        num_subcores=NS),
    scratch_types=(
        [
            pltpu.VMEM((NIDX, CH), jnp.int32),
            pltpu.VMEM((NIDX, CH), jnp.int32),
            pltpu.VMEM((NBUF, CH, F), jnp.float32),
            pltpu.VMEM_SHARED((N + 64, F), jnp.float32),
        ]
        + [pltpu.SemaphoreType.DMA] * (NIDX + 2 * NBUF)
    ),
)(_sc_agg_body)


def _mm_body(x_ref, w_ref, o_ref):
    o_ref[...] = jnp.dot(x_ref[...], w_ref[...],
                         preferred_element_type=jnp.float32)


def _support1(x, W):
    return pl.pallas_call(
        _mm_body,
        grid=(N // BN,),
        in_specs=[
            pl.BlockSpec((BN, F), lambda i: (i, 0)),
            pl.BlockSpec((F, F), lambda i: (0, 0)),
        ],
        out_specs=pl.BlockSpec((BN, F), lambda i: (i, 0)),
        out_shape=jax.ShapeDtypeStruct((N, F), jnp.float32),
    )(x, W)


def _gate_body(p_ref, q_ref, deg_ref, b1_ref, wg_ref, bg_ref, x_ref, w2_ref,
               g_ref, x1_ref, s2_ref):
    agg = (p_ref[0] + q_ref[0]) / deg_ref[...] + b1_ref[...]
    z = jnp.tanh(agg)
    g = jax.nn.sigmoid(jnp.dot(z, wg_ref[...],
                               preferred_element_type=jnp.float32)
                       + bg_ref[...])
    x1 = (1.0 - g) * x_ref[...] + g * z
    g_ref[...] = g
    x1_ref[...] = x1
    s2_ref[...] = jnp.dot(x1, w2_ref[...], preferred_element_type=jnp.float32)


def _gate_stage(parts, deg, b1, wg, bg, x, W2):
    return pl.pallas_call(
        _gate_body,
        grid=(N // BN,),
        in_specs=[
            pl.BlockSpec((1, BN, F), lambda i: (0, i, 0)),
            pl.BlockSpec((1, BN, F), lambda i: (1, i, 0)),
            pl.BlockSpec((BN, 1), lambda i: (i, 0)),
            pl.BlockSpec((1, F), lambda i: (0, 0)),
            pl.BlockSpec((F, 1), lambda i: (0, 0)),
            pl.BlockSpec((1, 1), lambda i: (0, 0)),
            pl.BlockSpec((BN, F), lambda i: (i, 0)),
            pl.BlockSpec((F, F), lambda i: (0, 0)),
        ],
        out_specs=[
            pl.BlockSpec((BN, 1), lambda i: (i, 0)),
            pl.BlockSpec((BN, F), lambda i: (i, 0)),
            pl.BlockSpec((BN, F), lambda i: (i, 0)),
        ],
        out_shape=[
            jax.ShapeDtypeStruct((N, 1), jnp.float32),
            jax.ShapeDtypeStruct((N, F), jnp.float32),
            jax.ShapeDtypeStruct((N, F), jnp.float32),
        ],
    )(parts, parts, deg, b1, wg, bg, x, W2)


def _final_body(p_ref, q_ref, deg_ref, b2_ref, wg_ref, bg_ref, x_ref,
                mu_ref, isg_ref, beta_ref, wo_ref, bo_ref,
                g_ref, out_ref):
    agg = (p_ref[0] + q_ref[0]) / deg_ref[...] + b2_ref[...]
    z = jnp.tanh(agg)
    g = jax.nn.sigmoid(jnp.dot(z, wg_ref[...],
                               preferred_element_type=jnp.float32)
                       + bg_ref[...])
    x2 = (1.0 - g) * x_ref[...] + g * z
    x2 = jnp.maximum(x2, 0.0)
    xb = (x2 - mu_ref[...]) * isg_ref[...] + beta_ref[...]
    g_ref[...] = g
    out_ref[...] = jnp.dot(xb, wo_ref[...],
                           preferred_element_type=jnp.float32) + bo_ref[...]


def _final_stage(parts, deg, b2, wg, bg, x1, mu, isg, beta, Wo, bo):
    return pl.pallas_call(
        _final_body,
        grid=(N // BN,),
        in_specs=[
            pl.BlockSpec((1, BN, F), lambda i: (0, i, 0)),
            pl.BlockSpec((1, BN, F), lambda i: (1, i, 0)),
            pl.BlockSpec((BN, 1), lambda i: (i, 0)),
            pl.BlockSpec((1, F), lambda i: (0, 0)),
            pl.BlockSpec((F, 1), lambda i: (0, 0)),
            pl.BlockSpec((1, 1), lambda i: (0, 0)),
            pl.BlockSpec((BN, F), lambda i: (i, 0)),
            pl.BlockSpec((1, F), lambda i: (0, 0)),
            pl.BlockSpec((1, F), lambda i: (0, 0)),
            pl.BlockSpec((1, F), lambda i: (0, 0)),
            pl.BlockSpec((F, NCLS), lambda i: (0, 0)),
            pl.BlockSpec((1, NCLS), lambda i: (0, 0)),
        ],
        out_specs=[
            pl.BlockSpec((BN, 1), lambda i: (i, 0)),
            pl.BlockSpec((BN, NCLS), lambda i: (i, 0)),
        ],
        out_shape=[
            jax.ShapeDtypeStruct((N, 1), jnp.float32),
            jax.ShapeDtypeStruct((N, NCLS), jnp.float32),
        ],
    )(parts, parts, deg, b2, wg, bg, x1, mu, isg, beta, Wo, bo)


def kernel(x_in, edge_index, deg, W_gc1, b_gc1, w_g1, b_g1, W_gc2, b_gc2,
           w_g2, b_g2, bn_gamma, bn_beta, bn_mean, bn_var, W_out, b_out):
    src = jnp.concatenate(
        [edge_index[0], jnp.zeros((EP - E,), jnp.int32)]).reshape(NW, CPW, CH)
    trash = N + (jnp.arange(EP - E, dtype=jnp.int32) % 64)
    dst = jnp.concatenate([edge_index[1], trash]).reshape(NW, CPW, CH)
    zeros = jnp.zeros((BR, F), jnp.float32)

    support1 = _support1(x_in, W_gc1)
    parts1 = _sc_agg(support1, src, dst, zeros)
    g, x1, support2 = _gate_stage(
        parts1, deg, b_gc1.reshape(1, F), w_g1, b_g1.reshape(1, 1),
        x_in, W_gc2)

    parts2 = _sc_agg(support2, src, dst, zeros)
    inv_sigma = (bn_gamma / jnp.sqrt(bn_var + 1e-5)).reshape(1, F)
    g2, out = _final_stage(
        parts2, deg, b_gc2.reshape(1, F), w_g2, b_g2.reshape(1, 1),
        x1, bn_mean.reshape(1, F), inv_sigma, bn_beta.reshape(1, F),
        W_out, b_out.reshape(1, NCLS))

    return (x_in, out, g, g2)


# speedup vs baseline: 1.2167x; 1.0943x over previous
import functools

import jax
import jax.numpy as jnp
from jax import lax
from jax.experimental import pallas as pl
from jax.experimental.pallas import tpu as pltpu
from jax.experimental.pallas import tpu_sc as plsc

N = 10000
F = 128
E = 320000
NCLS = 919

NC = 2
NS = 16
NW = NC * NS
CH = 64
CPW = 160
EP = NW * CPW * CH
NBUF = 5
NIDX = 2 * NBUF
NGRP = CPW // NIDX
BR = 80
NB = N // BR
BPT = -(-NB // NS)

BN = 1000


def _sc_agg_body(support_hbm, src_hbm, dst_hbm, zeros_hbm, out_hbm,
                 idx_s, idx_d, rows_v, acc, *sems):
    isems = sems[:NIDX]
    gsems = sems[NIDX:NIDX + NBUF]
    ssems = sems[NIDX + NBUF:]
    c = lax.axis_index("c")
    s = lax.axis_index("s")
    wid = c * NS + s

    def idx_start(chunk, q):
        pltpu.async_copy(src_hbm.at[wid, chunk], idx_s.at[q], isems[q])
        pltpu.async_copy(dst_hbm.at[wid, chunk], idx_d.at[q], isems[q])

    def idx_wait(q):
        pltpu.make_async_copy(src_hbm.at[wid, 0], idx_s.at[q],
                              isems[q]).wait()
        pltpu.make_async_copy(dst_hbm.at[wid, 0], idx_d.at[q],
                              isems[q]).wait()

    def gather_start(q, b):
        pltpu.async_copy(support_hbm.at[idx_s.at[q]], rows_v.at[b], gsems[b])

    def gather_wait(q, b):
        pltpu.make_async_copy(support_hbm.at[idx_s.at[q]], rows_v.at[b],
                              gsems[b]).wait()

    def scatter_start(q, b):
        pltpu.async_copy(rows_v.at[b], acc.at[idx_d.at[q]], ssems[b],
                         add=True)

    def scatter_wait(q, b):
        pltpu.make_async_copy(rows_v.at[b], acc.at[idx_d.at[q]],
                              ssems[b]).wait()

    for k in range(NIDX - 2):
        idx_start(k, k)

    def zero_blk(i, _):
        j = i * NS + s

        @pl.when(j < NB)
        def _():
            pltpu.sync_copy(zeros_hbm, acc.at[pl.ds(j * BR, BR), :])
        return 0

    lax.fori_loop(0, BPT, zero_blk, 0)
    plsc.subcore_barrier()

    idx_wait(0)
    gather_start(0, 0)
    idx_wait(1)
    gather_start(1, 1)

    def outer(i0, _):
        for k in range(NIDX):
            i = i0 * NIDX + k

            gather_wait(k, k % NBUF)
            scatter_start(k, k % NBUF)

            @pl.when(i >= 2)
            def _():
                scatter_wait((k - 2) % NIDX, (k - 2) % NBUF)

            @pl.when(i + 2 < CPW)
            def _():
                idx_wait((k + 2) % NIDX)
                gather_start((k + 2) % NIDX, (k + 2) % NBUF)

            @pl.when(i + NIDX - 2 < CPW)
            def _():
                idx_start(i + NIDX - 2, (k - 2) % NIDX)
        return 0

    lax.fori_loop(0, NGRP, outer, 0)

    for j in range(CPW - 2, CPW):
        scatter_wait(j % NIDX, j % NBUF)
    plsc.subcore_barrier()

    def drain_blk(i, _):
        j = i * NS + s

        @pl.when(j < NB)
        def _():
            pltpu.sync_copy(acc.at[pl.ds(j * BR, BR), :],
                            out_hbm.at[c, pl.ds(j * BR, BR), :])
        return 0

    lax.fori_loop(0, BPT, drain_blk, 0)


_sc_agg = functools.partial(
    pl.kernel,
    out_type=jax.ShapeDtypeStruct((NC, N, F), jnp.float32),
    mesh=plsc.VectorSubcoreMesh(
        core_axis_name="c", subcore_axis_name="s", num_cores=NC,
        num_subcores=NS),
    scratch_types=(
        [
            pltpu.VMEM((NIDX, CH), jnp.int32),
            pltpu.VMEM((NIDX, CH), jnp.int32),
            pltpu.VMEM((NBUF, CH, F), jnp.float32),
            pltpu.VMEM_SHARED((N + 64, F), jnp.float32),
        ]
        + [pltpu.SemaphoreType.DMA] * (NIDX + 2 * NBUF)
    ),
)(_sc_agg_body)


def _mm_body(x_ref, w_ref, o_ref):
    o_ref[...] = jnp.dot(x_ref[...], w_ref[...],
                         preferred_element_type=jnp.float32)


def _support1(x, W):
    return pl.pallas_call(
        _mm_body,
        grid=(N // BN,),
        in_specs=[
            pl.BlockSpec((BN, F), lambda i: (i, 0)),
            pl.BlockSpec((F, F), lambda i: (0, 0)),
        ],
        out_specs=pl.BlockSpec((BN, F), lambda i: (i, 0)),
        out_shape=jax.ShapeDtypeStruct((N, F), jnp.float32),
    )(x, W)


def _gate_body(p_ref, q_ref, deg_ref, b1_ref, wg_ref, bg_ref, x_ref, w2_ref,
               g_ref, x1_ref, s2_ref):
    agg = (p_ref[0] + q_ref[0]) / deg_ref[...] + b1_ref[...]
    z = jnp.tanh(agg)
    g = jax.nn.sigmoid(jnp.dot(z, wg_ref[...],
                               preferred_element_type=jnp.float32)
                       + bg_ref[...])
    x1 = (1.0 - g) * x_ref[...] + g * z
    g_ref[...] = g
    x1_ref[...] = x1
    s2_ref[...] = jnp.dot(x1, w2_ref[...], preferred_element_type=jnp.float32)


def _gate_stage(parts, deg, b1, wg, bg, x, W2):
    return pl.pallas_call(
        _gate_body,
        grid=(N // BN,),
        in_specs=[
            pl.BlockSpec((1, BN, F), lambda i: (0, i, 0)),
            pl.BlockSpec((1, BN, F), lambda i: (1, i, 0)),
            pl.BlockSpec((BN, 1), lambda i: (i, 0)),
            pl.BlockSpec((1, F), lambda i: (0, 0)),
            pl.BlockSpec((F, 1), lambda i: (0, 0)),
            pl.BlockSpec((1, 1), lambda i: (0, 0)),
            pl.BlockSpec((BN, F), lambda i: (i, 0)),
            pl.BlockSpec((F, F), lambda i: (0, 0)),
        ],
        out_specs=[
            pl.BlockSpec((BN, 1), lambda i: (i, 0)),
            pl.BlockSpec((BN, F), lambda i: (i, 0)),
            pl.BlockSpec((BN, F), lambda i: (i, 0)),
        ],
        out_shape=[
            jax.ShapeDtypeStruct((N, 1), jnp.float32),
            jax.ShapeDtypeStruct((N, F), jnp.float32),
            jax.ShapeDtypeStruct((N, F), jnp.float32),
        ],
    )(parts, parts, deg, b1, wg, bg, x, W2)


def _final_body(p_ref, q_ref, deg_ref, b2_ref, wg_ref, bg_ref, x_ref,
                mu_ref, isg_ref, beta_ref, wo_ref, bo_ref,
                g_ref, out_ref):
    agg = (p_ref[0] + q_ref[0]) / deg_ref[...] + b2_ref[...]
    z = jnp.tanh(agg)
    g = jax.nn.sigmoid(jnp.dot(z, wg_ref[...],
                               preferred_element_type=jnp.float32)
                       + bg_ref[...])
    x2 = (1.0 - g) * x_ref[...] + g * z
    x2 = jnp.maximum(x2, 0.0)
    xb = (x2 - mu_ref[...]) * isg_ref[...] + beta_ref[...]
    g_ref[...] = g
    out_ref[...] = jnp.dot(xb, wo_ref[...],
                           preferred_element_type=jnp.float32) + bo_ref[...]


def _final_stage(parts, deg, b2, wg, bg, x1, mu, isg, beta, Wo, bo):
    return pl.pallas_call(
        _final_body,
        grid=(N // BN,),
        in_specs=[
            pl.BlockSpec((1, BN, F), lambda i: (0, i, 0)),
            pl.BlockSpec((1, BN, F), lambda i: (1, i, 0)),
            pl.BlockSpec((BN, 1), lambda i: (i, 0)),
            pl.BlockSpec((1, F), lambda i: (0, 0)),
            pl.BlockSpec((F, 1), lambda i: (0, 0)),
            pl.BlockSpec((1, 1), lambda i: (0, 0)),
            pl.BlockSpec((BN, F), lambda i: (i, 0)),
            pl.BlockSpec((1, F), lambda i: (0, 0)),
            pl.BlockSpec((1, F), lambda i: (0, 0)),
            pl.BlockSpec((1, F), lambda i: (0, 0)),
            pl.BlockSpec((F, NCLS), lambda i: (0, 0)),
            pl.BlockSpec((1, NCLS), lambda i: (0, 0)),
        ],
        out_specs=[
            pl.BlockSpec((BN, 1), lambda i: (i, 0)),
            pl.BlockSpec((BN, NCLS), lambda i: (i, 0)),
        ],
        out_shape=[
            jax.ShapeDtypeStruct((N, 1), jnp.float32),
            jax.ShapeDtypeStruct((N, NCLS), jnp.float32),
        ],
    )(parts, parts, deg, b2, wg, bg, x1, mu, isg, beta, Wo, bo)


def kernel(x_in, edge_index, deg, W_gc1, b_gc1, w_g1, b_g1, W_gc2, b_gc2,
           w_g2, b_g2, bn_gamma, bn_beta, bn_mean, bn_var, W_out, b_out):
    src = jnp.concatenate(
        [edge_index[0], jnp.zeros((EP - E,), jnp.int32)]).reshape(NW, CPW, CH)
    trash = N + (jnp.arange(EP - E, dtype=jnp.int32) % 64)
    dst = jnp.concatenate([edge_index[1], trash]).reshape(NW, CPW, CH)
    zeros = jnp.zeros((BR, F), jnp.float32)

    support1 = _support1(x_in, W_gc1)
    parts1 = _sc_agg(support1, src, dst, zeros)
    g, x1, support2 = _gate_stage(
        parts1, deg, b_gc1.reshape(1, F), w_g1, b_g1.reshape(1, 1),
        x_in, W_gc2)

    parts2 = _sc_agg(support2, src, dst, zeros)
    inv_sigma = (bn_gamma / jnp.sqrt(bn_var + 1e-5)).reshape(1, F)
    g2, out = _final_stage(
        parts2, deg, b_gc2.reshape(1, F), w_g2, b_g2.reshape(1, 1),
        x1, bn_mean.reshape(1, F), inv_sigma, bn_beta.reshape(1, F),
        W_out, b_out.reshape(1, NCLS))

    return (x_in, out, g, g2)


# speedup vs baseline: 2.1737x; 1.7865x over previous
import functools

import jax
import jax.numpy as jnp
from jax import lax
from jax.experimental import pallas as pl
from jax.experimental.pallas import tpu as pltpu
from jax.experimental.pallas import tpu_sc as plsc

N = 10000
F = 128
E = 320000
NCLS = 919

NC = 2
NS = 16
NW = NC * NS
CH = 80
CPW = 126
EP = NW * CPW * CH
NBUF = 3
NIDX = 6
NGRP = CPW // NIDX
BR = 80
NB = N // BR
BPT = -(-NB // NS)

BN = 1000

DIAG = "pipeline"
LOOKAHEAD = 2


def _sc_agg_body(support_hbm, src_hbm, dst_hbm, zeros_hbm, out_hbm, *scr):
    idx_s = scr[:NIDX]
    idx_d = scr[NIDX:2 * NIDX]
    rows_v = scr[2 * NIDX:2 * NIDX + NBUF]
    acc = scr[2 * NIDX + NBUF]
    sems = scr[2 * NIDX + NBUF + 1:]
    isems = sems[:NIDX]
    gsems = sems[NIDX:NIDX + NBUF]
    ssems = sems[NIDX + NBUF:]
    c = lax.axis_index("c")
    s = lax.axis_index("s")
    wid = c * NS + s

    def idx_start(chunk, q):
        pltpu.async_copy(src_hbm.at[wid, chunk], idx_s[q], isems[q])
        pltpu.async_copy(dst_hbm.at[wid, chunk], idx_d[q], isems[q])

    def idx_wait(q):
        pltpu.make_async_copy(src_hbm.at[wid, 0], idx_s[q],
                              isems[q]).wait()
        pltpu.make_async_copy(dst_hbm.at[wid, 0], idx_d[q],
                              isems[q]).wait()

    def gather_start(q, b):
        if DIAG in ("sync_gather", "sync_gather_only"):
            return
        if DIAG != "scatter_only":
            pltpu.async_copy(support_hbm.at[idx_s[q]], rows_v[b],
                             gsems[b])

    def gather_wait(q, b):
        if DIAG in ("sync_gather", "sync_gather_only"):
            pltpu.sync_copy(support_hbm.at[idx_s[q]], rows_v[b])
            return
        if DIAG != "scatter_only":
            pltpu.make_async_copy(support_hbm.at[idx_s[q]], rows_v[b],
                                  gsems[b]).wait()

    def scatter_start(q, b):
        if DIAG not in ("gather_only", "sync_gather_only"):
            pltpu.async_copy(rows_v[b], acc.at[idx_d[q]], ssems[b],
                             add=True)

    def scatter_wait(q, b):
        if DIAG not in ("gather_only", "sync_gather_only"):
            pltpu.make_async_copy(rows_v[b], acc.at[idx_d[q]],
                                  ssems[b]).wait()

    if DIAG != "serial":
        for k in range(NIDX - 2):
            idx_start(k, k)

    def zero_blk(i, _):
        j = i * NS + s

        @pl.when(j < NB)
        def _():
            pltpu.sync_copy(zeros_hbm, acc.at[pl.ds(j * BR, BR), :])
        return 0

    lax.fori_loop(0, BPT, zero_blk, 0)
    plsc.subcore_barrier()

    if DIAG == "serial":
        def body(i, _):
            pltpu.sync_copy(src_hbm.at[wid, i], idx_s[0])
            pltpu.sync_copy(dst_hbm.at[wid, i], idx_d[0])
            pltpu.async_copy(support_hbm.at[idx_s[0]], rows_v[0],
                             gsems[0]).wait()
            pltpu.sync_copy(rows_v[0], acc.at[idx_d[0]], add=True)
            return 0

        lax.fori_loop(0, CPW, body, 0)
        plsc.subcore_barrier()

        def drain_blk2(i, _):
            j = i * NS + s

            @pl.when(j < NB)
            def _():
                pltpu.sync_copy(acc.at[pl.ds(j * BR, BR), :],
                                out_hbm.at[c, pl.ds(j * BR, BR), :])
            return 0

        lax.fori_loop(0, BPT, drain_blk2, 0)
        return

    for k in range(LOOKAHEAD):
        idx_wait(k)
        gather_start(k, k % NBUF)

    def outer(i0, _):
        for k in range(NIDX):
            i = i0 * NIDX + k

            gather_wait(k, k % NBUF)
            scatter_start(k, k % NBUF)

            @pl.when(i >= 1)
            def _():
                scatter_wait((k - 1) % NIDX, (k - 1) % NBUF)

            @pl.when(i + LOOKAHEAD < CPW)
            def _():
                idx_wait((k + LOOKAHEAD) % NIDX)
                gather_start((k + LOOKAHEAD) % NIDX, (k + LOOKAHEAD) % NBUF)

            @pl.when(i + NIDX - 2 < CPW)
            def _():
                idx_start(i + NIDX - 2, (k - 2) % NIDX)
        return 0

    lax.fori_loop(0, NGRP, outer, 0)

    scatter_wait((CPW - 1) % NIDX, (CPW - 1) % NBUF)
    plsc.subcore_barrier()

    def drain_blk(i, _):
        j = i * NS + s

        @pl.when(j < NB)
        def _():
            pltpu.sync_copy(acc.at[pl.ds(j * BR, BR), :],
                            out_hbm.at[c, pl.ds(j * BR, BR), :])
        return 0

    lax.fori_loop(0, BPT, drain_blk, 0)


_sc_agg = functools.partial(
    pl.kernel,
    out_type=jax.ShapeDtypeStruct((NC, N, F), jnp.float32),
    mesh=plsc.VectorSubcoreMesh(
        core_axis_name="c", subcore_axis_name="s", num_cores=NC,
        num_subcores=NS),
    scratch_types=(
        [pltpu.VMEM((CH,), jnp.int32)] * (2 * NIDX)
        + [pltpu.VMEM((CH, F), jnp.float32)] * NBUF
        + [pltpu.VMEM_SHARED((N + 64, F), jnp.float32)]
        + [pltpu.SemaphoreType.DMA] * (NIDX + 2 * NBUF)
    ),
)(_sc_agg_body)


def _mm_body(x_ref, w_ref, o_ref):
    o_ref[...] = jnp.dot(x_ref[...], w_ref[...],
                         preferred_element_type=jnp.float32)


def _support1(x, W):
    return pl.pallas_call(
        _mm_body,
        grid=(N // BN,),
        in_specs=[
            pl.BlockSpec((BN, F), lambda i: (i, 0)),
            pl.BlockSpec((F, F), lambda i: (0, 0)),
        ],
        out_specs=pl.BlockSpec((BN, F), lambda i: (i, 0)),
        out_shape=jax.ShapeDtypeStruct((N, F), jnp.float32),
    )(x, W)


def _gate_body(p_ref, q_ref, deg_ref, b1_ref, wg_ref, bg_ref, x_ref, w2_ref,
               g_ref, x1_ref, s2_ref):
    agg = (p_ref[0] + q_ref[0]) / deg_ref[...] + b1_ref[...]
    z = jnp.tanh(agg)
    g = jax.nn.sigmoid(jnp.dot(z, wg_ref[...],
                               preferred_element_type=jnp.float32)
                       + bg_ref[...])
    x1 = (1.0 - g) * x_ref[...] + g * z
    g_ref[...] = g
    x1_ref[...] = x1
    s2_ref[...] = jnp.dot(x1, w2_ref[...], preferred_element_type=jnp.float32)


def _gate_stage(parts, deg, b1, wg, bg, x, W2):
    return pl.pallas_call(
        _gate_body,
        grid=(N // BN,),
        in_specs=[
            pl.BlockSpec((1, BN, F), lambda i: (0, i, 0)),
            pl.BlockSpec((1, BN, F), lambda i: (1, i, 0)),
            pl.BlockSpec((BN, 1), lambda i: (i, 0)),
            pl.BlockSpec((1, F), lambda i: (0, 0)),
            pl.BlockSpec((F, 1), lambda i: (0, 0)),
            pl.BlockSpec((1, 1), lambda i: (0, 0)),
            pl.BlockSpec((BN, F), lambda i: (i, 0)),
            pl.BlockSpec((F, F), lambda i: (0, 0)),
        ],
        out_specs=[
            pl.BlockSpec((BN, 1), lambda i: (i, 0)),
            pl.BlockSpec((BN, F), lambda i: (i, 0)),
            pl.BlockSpec((BN, F), lambda i: (i, 0)),
        ],
        out_shape=[
            jax.ShapeDtypeStruct((N, 1), jnp.float32),
            jax.ShapeDtypeStruct((N, F), jnp.float32),
            jax.ShapeDtypeStruct((N, F), jnp.float32),
        ],
    )(parts, parts, deg, b1, wg, bg, x, W2)


def _final_body(p_ref, q_ref, deg_ref, b2_ref, wg_ref, bg_ref, x_ref,
                mu_ref, isg_ref, beta_ref, wo_ref, bo_ref,
                g_ref, out_ref):
    agg = (p_ref[0] + q_ref[0]) / deg_ref[...] + b2_ref[...]
    z = jnp.tanh(agg)
    g = jax.nn.sigmoid(jnp.dot(z, wg_ref[...],
                               preferred_element_type=jnp.float32)
                       + bg_ref[...])
    x2 = (1.0 - g) * x_ref[...] + g * z
    x2 = jnp.maximum(x2, 0.0)
    xb = (x2 - mu_ref[...]) * isg_ref[...] + beta_ref[...]
    g_ref[...] = g
    out_ref[...] = jnp.dot(xb, wo_ref[...],
                           preferred_element_type=jnp.float32) + bo_ref[...]


def _final_stage(parts, deg, b2, wg, bg, x1, mu, isg, beta, Wo, bo):
    return pl.pallas_call(
        _final_body,
        grid=(N // BN,),
        in_specs=[
            pl.BlockSpec((1, BN, F), lambda i: (0, i, 0)),
            pl.BlockSpec((1, BN, F), lambda i: (1, i, 0)),
            pl.BlockSpec((BN, 1), lambda i: (i, 0)),
            pl.BlockSpec((1, F), lambda i: (0, 0)),
            pl.BlockSpec((F, 1), lambda i: (0, 0)),
            pl.BlockSpec((1, 1), lambda i: (0, 0)),
            pl.BlockSpec((BN, F), lambda i: (i, 0)),
            pl.BlockSpec((1, F), lambda i: (0, 0)),
            pl.BlockSpec((1, F), lambda i: (0, 0)),
            pl.BlockSpec((1, F), lambda i: (0, 0)),
            pl.BlockSpec((F, NCLS), lambda i: (0, 0)),
            pl.BlockSpec((1, NCLS), lambda i: (0, 0)),
        ],
        out_specs=[
            pl.BlockSpec((BN, 1), lambda i: (i, 0)),
            pl.BlockSpec((BN, NCLS), lambda i: (i, 0)),
        ],
        out_shape=[
            jax.ShapeDtypeStruct((N, 1), jnp.float32),
            jax.ShapeDtypeStruct((N, NCLS), jnp.float32),
        ],
    )(parts, parts, deg, b2, wg, bg, x1, mu, isg, beta, Wo, bo)


def kernel(x_in, edge_index, deg, W_gc1, b_gc1, w_g1, b_g1, W_gc2, b_gc2,
           w_g2, b_g2, bn_gamma, bn_beta, bn_mean, bn_var, W_out, b_out):
    src = jnp.concatenate(
        [edge_index[0], jnp.zeros((EP - E,), jnp.int32)]).reshape(NW, CPW, CH)
    trash = N + (jnp.arange(EP - E, dtype=jnp.int32) % 64)
    dst = jnp.concatenate([edge_index[1], trash]).reshape(NW, CPW, CH)
    zeros = jnp.zeros((BR, F), jnp.float32)

    support1 = _support1(x_in, W_gc1)
    parts1 = _sc_agg(support1, src, dst, zeros)
    g, x1, support2 = _gate_stage(
        parts1, deg, b_gc1.reshape(1, F), w_g1, b_g1.reshape(1, 1),
        x_in, W_gc2)

    parts2 = _sc_agg(support2, src, dst, zeros)
    inv_sigma = (bn_gamma / jnp.sqrt(bn_var + 1e-5)).reshape(1, F)
    g2, out = _final_stage(
        parts2, deg, b_gc2.reshape(1, F), w_g2, b_g2.reshape(1, 1),
        x1, bn_mean.reshape(1, F), inv_sigma, bn_beta.reshape(1, F),
        W_out, b_out.reshape(1, NCLS))

    return (x_in, out, g, g2)
